# baseline (speedup 1.0000x reference)
import functools

import jax
import jax.numpy as jnp
from jax import lax
from jax.experimental import pallas as pl
from jax.experimental.pallas import tpu as pltpu
from jax.experimental.pallas import tpu_sc as plsc

D = 64
SI = 10
NBUF = 4


@functools.cache
def _make_gather(BATCH, S, W):
    info = plsc.get_sparse_core_info()
    nw = info.num_cores * info.num_subcores
    B = BATCH * S * W
    b_per_w = BATCH // nw
    cpb = S // SI
    n_chunks = b_per_w * cpb
    CH = SI * W
    assert S % SI == 0 and BATCH % nw == 0 and n_chunks % NBUF == 0

    mesh = plsc.VectorSubcoreMesh(core_axis_name="c", subcore_axis_name="s")

    @functools.partial(
        pl.kernel,
        mesh=mesh,
        out_type=jax.ShapeDtypeStruct((B, 2 * D), jnp.float32),
        scratch_types=[
            pltpu.VMEM((b_per_w, S, W), jnp.int32),
            pltpu.VMEM((NBUF, CH, D), jnp.float32),
        ]
        + [pltpu.SemaphoreType.DMA] * (2 * NBUF),
        compiler_params=pltpu.CompilerParams(use_tc_tiling_on_sc=False),
    )
    def emb(x_hbm, table_hbm, out_hbm, idx_all, rows, *sems):
        sem_g, sem_w = sems[:NBUF], sems[NBUF:]
        wid = lax.axis_index("s") * info.num_cores + lax.axis_index("c")
        b0 = wid * b_per_w

        pltpu.sync_copy(x_hbm.at[pl.ds(b0, b_per_w)], idx_all)

        def fire_gather(c, slot):
            bi = c // cpb
            si0 = (c % cpb) * SI
            for t in range(SI):
                pltpu.async_copy(
                    table_hbm.at[idx_all.at[bi, si0 + t]],
                    rows.at[slot].at[pl.ds(t * W, W)],
                    sem_g[slot],
                )

        def wait_gather(slot):
            pltpu.make_async_copy(
                table_hbm.at[pl.ds(0, CH)], rows.at[slot], sem_g[slot]
            ).wait()

        def _write_copy(c, slot):
            flat0 = (b0 + c // cpb) * S * W + (c % cpb) * SI * W
            return pltpu.make_async_copy(
                rows.at[slot],
                out_hbm.at[pl.ds(flat0, CH), pl.ds(0, D)],
                sem_w[slot],
            )

        def fire_write(c, slot):
            _write_copy(c, slot).start()

        def wait_write(c, slot):
            _write_copy(c, slot).wait()

        fire_gather(0, 0)
        fire_gather(1, 1)

        def group(t, carry):
            for b in range(NBUF):
                c = t * NBUF + b
                wait_gather(b)
                fire_write(c, b)
                s2 = (b + 2) % NBUF

                @pl.when(c + 2 < n_chunks)
                def _():
                    @pl.when(c >= 2)
                    def _():
                        wait_write(c - 2, s2)

                    fire_gather(c + 2, s2)

            return carry

        lax.fori_loop(0, n_chunks // NBUF, group, 0)

        for b in range(NBUF):
            wait_write(n_chunks - NBUF + b, b)

    return emb


def kernel(x, table):
    BATCH, S, W = x.shape
    out = _make_gather(BATCH, S, W)(x.astype(jnp.int32), table)
    return out.reshape(BATCH, S, W, 2 * D)[..., :D]


# speedup vs baseline: 1.0034x; 1.0034x over previous
import functools

import jax
import jax.numpy as jnp
from jax import lax
from jax.experimental import pallas as pl
from jax.experimental.pallas import tpu as pltpu
from jax.experimental.pallas import tpu_sc as plsc

D = 64
SI = 10
NBUF = 5
PRE = 3


@functools.cache
def _make_gather(BATCH, S, W):
    info = plsc.get_sparse_core_info()
    nw = info.num_cores * info.num_subcores
    B = BATCH * S * W
    b_per_w = BATCH // nw
    cpb = S // SI
    n_chunks = b_per_w * cpb
    CH = SI * W
    assert S % SI == 0 and BATCH % nw == 0 and n_chunks % NBUF == 0

    mesh = plsc.VectorSubcoreMesh(core_axis_name="c", subcore_axis_name="s")

    @functools.partial(
        pl.kernel,
        mesh=mesh,
        out_type=jax.ShapeDtypeStruct((B, 2 * D), jnp.float32),
        scratch_types=[
            pltpu.VMEM((b_per_w, S, W), jnp.int32),
            pltpu.VMEM((NBUF, CH, D), jnp.float32),
        ]
        + [pltpu.SemaphoreType.DMA] * (2 * NBUF),
        compiler_params=pltpu.CompilerParams(use_tc_tiling_on_sc=False),
    )
    def emb(x_hbm, table_hbm, out_hbm, idx_all, rows, *sems):
        sem_g, sem_w = sems[:NBUF], sems[NBUF:]
        wid = lax.axis_index("s") * info.num_cores + lax.axis_index("c")
        b0 = wid * b_per_w

        pltpu.sync_copy(x_hbm.at[pl.ds(b0, b_per_w)], idx_all)

        def fire_gather(c, slot):
            bi = c // cpb
            si0 = (c % cpb) * SI
            for t in range(SI):
                pltpu.async_copy(
                    table_hbm.at[idx_all.at[bi, si0 + t]],
                    rows.at[slot].at[pl.ds(t * W, W)],
                    sem_g[slot],
                )

        def wait_gather(slot):
            pltpu.make_async_copy(
                table_hbm.at[pl.ds(0, CH)], rows.at[slot], sem_g[slot]
            ).wait()

        def _write_copy(c, slot):
            flat0 = (b0 + c // cpb) * S * W + (c % cpb) * SI * W
            return pltpu.make_async_copy(
                rows.at[slot],
                out_hbm.at[pl.ds(flat0, CH), pl.ds(0, D)],
                sem_w[slot],
            )

        def fire_write(c, slot):
            _write_copy(c, slot).start()

        def wait_write(c, slot):
            _write_copy(c, slot).wait()

        for c0 in range(PRE):
            fire_gather(c0, c0)

        def group(t, carry):
            for b in range(NBUF):
                c = t * NBUF + b
                wait_gather(b)
                fire_write(c, b)
                s2 = (b + PRE) % NBUF

                @pl.when(c + PRE < n_chunks)
                def _():
                    @pl.when(c >= NBUF - PRE)
                    def _():
                        wait_write(c - (NBUF - PRE), s2)

                    fire_gather(c + PRE, s2)

            return carry

        lax.fori_loop(0, n_chunks // NBUF, group, 0)

        for b in range(NBUF):
            wait_write(n_chunks - NBUF + b, b)

    return emb


def kernel(x, table):
    BATCH, S, W = x.shape
    out = _make_gather(BATCH, S, W)(x.astype(jnp.int32), table)
    return out.reshape(BATCH, S, W, 2 * D)[..., :D]
